# baseline (speedup 1.0000x reference)
import jax
import jax.numpy as jnp
from jax.experimental import pallas as pl
from jax.experimental.pallas import tpu as pltpu

N = 4096
L = 64
STEP_DIM = 2048
COMB_DIM = 1024
BK = 512
NK = 2 * STEP_DIM // BK
JH = STEP_DIM // BK


def _step_kernel(node_sref, adj_row_ref, feats_ref, labels_ref, att_ref,
                 scores_ref, u_ref, t1_ref, t2_ref, t3_ref,
                 state_ref, newnode_ref, attscore_ref, comb_ref):
    k = pl.program_id(0)

    @pl.when(k == 0)
    def _sampling():
        adj_row = adj_row_ref[0, 0, :]
        feats = feats_ref[0, :]
        att = att_ref[0, :]

        nf = jnp.sum(adj_row * feats)
        spread = att * nf
        s = jnp.sum(spread)
        norm = spread / s

        row_ids = jax.lax.broadcasted_iota(jnp.int32, (L, L), 0)
        col_ids = jax.lax.broadcasted_iota(jnp.int32, (L, L), 1)
        tri = (row_ids <= col_ids).astype(jnp.float32)
        cum = jnp.dot(norm.reshape(1, L), tri,
                      preferred_element_type=jnp.float32)[0, :]
        r = cum[L - 1] * (1.0 - u_ref[0, 0])
        label = jnp.sum((cum < r).astype(jnp.int32)).astype(jnp.int32)

        cand = (adj_row > 0.0) & (labels_ref[0, :] == label)
        n_cand = jnp.sum(cand.astype(jnp.int32))
        sc = jnp.where(cand, scores_ref[0, :], -jnp.inf)
        m = jnp.max(sc)
        idx = jax.lax.broadcasted_iota(jnp.int32, (N,), 0)
        first_max = jnp.min(jnp.where(sc == m, idx, N))
        new_node = jnp.where(n_cand > 0, first_max, node_sref[0])
        newnode_ref[...] = new_node.astype(jnp.int32).reshape(1, 1)

        onehot = (jax.lax.broadcasted_iota(jnp.int32, (1, L), 1)
                  == label).astype(jnp.float32)
        attscore_ref[...] = jnp.sum(onehot[0, :] * att).reshape(1, 1)

        att_row = att.reshape(1, L)
        for j in range(JH):
            comb_ref[j:j + 1, :] = jnp.dot(
                att_row, t1_ref[:, j * BK:(j + 1) * BK],
                preferred_element_type=jnp.float32)
        for j in range(JH):
            comb_ref[JH + j:JH + j + 1, :] = jnp.dot(
                onehot, t2_ref[:, j * BK:(j + 1) * BK],
                preferred_element_type=jnp.float32)

    part = jnp.dot(comb_ref[pl.ds(k, 1), :], t3_ref[...],
                   preferred_element_type=jnp.float32)

    @pl.when(k == 0)
    def _init():
        state_ref[...] = part

    @pl.when(k > 0)
    def _acc():
        state_ref[...] += part


def kernel(adj, features, node_labels, node, attention,
           theta_step_1, theta_step_2, theta_step_3):
    with jax.ensure_compile_time_eval():
        key = jax.random.key(42)
        k1, k2 = jax.random.split(key)
        u1 = jax.random.uniform(k1, ())
        scores = jax.random.uniform(k2, (N,))

    node_arr = jnp.asarray(node, jnp.int32).reshape((1,))

    grid_spec = pltpu.PrefetchScalarGridSpec(
        num_scalar_prefetch=1,
        grid=(NK,),
        in_specs=[
            pl.BlockSpec((1, 1, N), lambda k, n: (n[0], 0, 0)),
            pl.BlockSpec((1, N), lambda k, n: (0, 0)),
            pl.BlockSpec((1, N), lambda k, n: (0, 0)),
            pl.BlockSpec((1, L), lambda k, n: (0, 0)),
            pl.BlockSpec((1, N), lambda k, n: (0, 0)),
            pl.BlockSpec((1, 1), lambda k, n: (0, 0)),
            pl.BlockSpec((L, STEP_DIM), lambda k, n: (0, 0)),
            pl.BlockSpec((L, STEP_DIM), lambda k, n: (0, 0)),
            pl.BlockSpec((BK, COMB_DIM), lambda k, n: (k, 0)),
        ],
        out_specs=[
            pl.BlockSpec((1, COMB_DIM), lambda k, n: (0, 0)),
            pl.BlockSpec((1, 1), lambda k, n: (0, 0)),
            pl.BlockSpec((1, 1), lambda k, n: (0, 0)),
        ],
        scratch_shapes=[pltpu.VMEM((NK, BK), jnp.float32)],
    )

    state, new_node, att_score = pl.pallas_call(
        _step_kernel,
        grid_spec=grid_spec,
        out_shape=[
            jax.ShapeDtypeStruct((1, COMB_DIM), jnp.float32),
            jax.ShapeDtypeStruct((1, 1), jnp.int32),
            jax.ShapeDtypeStruct((1, 1), jnp.float32),
        ],
    )(node_arr, adj.reshape(N, 1, N), features.reshape(1, N),
      node_labels.astype(jnp.int32).reshape(1, N),
      attention.reshape(1, L), scores.reshape(1, N), u1.reshape(1, 1),
      theta_step_1, theta_step_2, theta_step_3)

    return (state.reshape(1, 1, COMB_DIM),
            new_node.reshape(()),
            att_score.reshape(()))


# speedup vs baseline: 3.6299x; 3.6299x over previous
import jax
import jax.numpy as jnp
from jax.experimental import pallas as pl
from jax.experimental.pallas import tpu as pltpu

N = 4096
L = 64
STEP_DIM = 2048
COMB_DIM = 1024
BK = 512
NK = 2 * STEP_DIM // BK
JH = STEP_DIM // BK


def _step_kernel(node_sref, adj_row_ref, feats_ref, labels_ref, att_ref,
                 scores_ref, u_ref, t1_ref, t2_ref, t3_ref,
                 state_ref, newnode_ref, attscore_ref, comb_ref):
    k = pl.program_id(0)

    @pl.when(k == 0)
    def _sampling():
        adj_row = adj_row_ref[node_sref[0] % 8, :]
        feats = feats_ref[0, :]
        att = att_ref[0, :]

        nf = jnp.sum(adj_row * feats)
        spread = att * nf
        s = jnp.sum(spread)
        norm = spread / s

        row_ids = jax.lax.broadcasted_iota(jnp.int32, (L, L), 0)
        col_ids = jax.lax.broadcasted_iota(jnp.int32, (L, L), 1)
        tri = (row_ids <= col_ids).astype(jnp.float32)
        cum = jnp.dot(norm.reshape(1, L), tri,
                      preferred_element_type=jnp.float32)[0, :]
        r = cum[L - 1] * (1.0 - u_ref[0, 0])
        label = jnp.sum((cum < r).astype(jnp.int32)).astype(jnp.int32)

        cand = (adj_row > 0.0) & (labels_ref[0, :] == label)
        n_cand = jnp.sum(cand.astype(jnp.int32))
        sc = jnp.where(cand, scores_ref[0, :], -jnp.inf)
        m = jnp.max(sc)
        idx = jax.lax.broadcasted_iota(jnp.int32, (N,), 0)
        first_max = jnp.min(jnp.where(sc == m, idx, N))
        new_node = jnp.where(n_cand > 0, first_max, node_sref[0])
        newnode_ref[...] = new_node.astype(jnp.int32).reshape(1, 1)

        onehot = (jax.lax.broadcasted_iota(jnp.int32, (1, L), 1)
                  == label).astype(jnp.float32)
        attscore_ref[...] = jnp.sum(onehot[0, :] * att).reshape(1, 1)

        att_row = att.reshape(1, L)
        for j in range(JH):
            comb_ref[j:j + 1, :] = jnp.dot(
                att_row, t1_ref[:, j * BK:(j + 1) * BK],
                preferred_element_type=jnp.float32)
        for j in range(JH):
            comb_ref[JH + j:JH + j + 1, :] = jnp.dot(
                onehot, t2_ref[:, j * BK:(j + 1) * BK],
                preferred_element_type=jnp.float32)

    part = jnp.dot(comb_ref[pl.ds(k, 1), :], t3_ref[...],
                   preferred_element_type=jnp.float32)

    @pl.when(k == 0)
    def _init():
        state_ref[...] = part

    @pl.when(k > 0)
    def _acc():
        state_ref[...] += part


def kernel(adj, features, node_labels, node, attention,
           theta_step_1, theta_step_2, theta_step_3):
    key = jax.random.key(42)
    k1, k2 = jax.random.split(key)
    u1 = jax.random.uniform(k1, ())
    scores = jax.random.uniform(k2, (N,))

    node_arr = jnp.asarray(node, jnp.int32).reshape((1,))

    grid_spec = pltpu.PrefetchScalarGridSpec(
        num_scalar_prefetch=1,
        grid=(NK,),
        in_specs=[
            pl.BlockSpec((8, N), lambda k, n: (n[0] // 8, 0)),
            pl.BlockSpec((1, N), lambda k, n: (0, 0)),
            pl.BlockSpec((1, N), lambda k, n: (0, 0)),
            pl.BlockSpec((1, L), lambda k, n: (0, 0)),
            pl.BlockSpec((1, N), lambda k, n: (0, 0)),
            pl.BlockSpec((1, 1), lambda k, n: (0, 0)),
            pl.BlockSpec((L, STEP_DIM), lambda k, n: (0, 0)),
            pl.BlockSpec((L, STEP_DIM), lambda k, n: (0, 0)),
            pl.BlockSpec((BK, COMB_DIM), lambda k, n: (k, 0)),
        ],
        out_specs=[
            pl.BlockSpec((1, COMB_DIM), lambda k, n: (0, 0)),
            pl.BlockSpec((1, 1), lambda k, n: (0, 0)),
            pl.BlockSpec((1, 1), lambda k, n: (0, 0)),
        ],
        scratch_shapes=[pltpu.VMEM((NK, BK), jnp.float32)],
    )

    state, new_node, att_score = pl.pallas_call(
        _step_kernel,
        grid_spec=grid_spec,
        out_shape=[
            jax.ShapeDtypeStruct((1, COMB_DIM), jnp.float32),
            jax.ShapeDtypeStruct((1, 1), jnp.int32),
            jax.ShapeDtypeStruct((1, 1), jnp.float32),
        ],
    )(node_arr, adj, features.reshape(1, N),
      node_labels.astype(jnp.int32).reshape(1, N),
      attention.reshape(1, L), scores.reshape(1, N), u1.reshape(1, 1),
      theta_step_1, theta_step_2, theta_step_3)

    return (state.reshape(1, 1, COMB_DIM),
            new_node.reshape(()),
            att_score.reshape(()))


# speedup vs baseline: 3.8878x; 1.0710x over previous
import jax
import jax.numpy as jnp
from jax.experimental import pallas as pl
from jax.experimental.pallas import tpu as pltpu

N = 4096
L = 64
STEP_DIM = 2048
COMB_DIM = 1024
BK = 1024
NK = 2 * STEP_DIM // BK
JH = STEP_DIM // BK


def _step_kernel(node_sref, adj_row_ref, feats_ref, labels_ref, att_ref,
                 scores_ref, u_ref, t1_ref, t2_ref, t3_ref,
                 state_ref, newnode_ref, attscore_ref, comb_ref):
    k = pl.program_id(0)

    @pl.when(k == 0)
    def _sampling():
        adj_row = adj_row_ref[node_sref[0] % 8, :]
        feats = feats_ref[0, :]
        att = att_ref[0, :]

        nf = jnp.sum(adj_row * feats)
        spread = att * nf
        s = jnp.sum(spread)
        norm = spread / s

        row_ids = jax.lax.broadcasted_iota(jnp.int32, (L, L), 0)
        col_ids = jax.lax.broadcasted_iota(jnp.int32, (L, L), 1)
        tri = (row_ids <= col_ids).astype(jnp.float32)
        cum = jnp.dot(norm.reshape(1, L), tri,
                      preferred_element_type=jnp.float32)[0, :]
        r = cum[L - 1] * (1.0 - u_ref[0, 0])
        label = jnp.sum((cum < r).astype(jnp.int32)).astype(jnp.int32)

        cand = (adj_row > 0.0) & (labels_ref[0, :] == label)
        n_cand = jnp.sum(cand.astype(jnp.int32))
        sc = jnp.where(cand, scores_ref[0, :], -jnp.inf)
        m = jnp.max(sc)
        idx = jax.lax.broadcasted_iota(jnp.int32, (N,), 0)
        first_max = jnp.min(jnp.where(sc == m, idx, N))
        new_node = jnp.where(n_cand > 0, first_max, node_sref[0])
        newnode_ref[...] = new_node.astype(jnp.int32).reshape(1, 1)

        onehot = (jax.lax.broadcasted_iota(jnp.int32, (1, L), 1)
                  == label).astype(jnp.float32)
        attscore_ref[...] = jnp.sum(onehot[0, :] * att).reshape(1, 1)

        att_row = att.reshape(1, L)
        for j in range(JH):
            comb_ref[j:j + 1, :] = jnp.dot(
                att_row, t1_ref[:, j * BK:(j + 1) * BK],
                preferred_element_type=jnp.float32)
        for j in range(JH):
            comb_ref[JH + j:JH + j + 1, :] = jnp.dot(
                onehot, t2_ref[:, j * BK:(j + 1) * BK],
                preferred_element_type=jnp.float32)

    part = jnp.dot(comb_ref[pl.ds(k, 1), :], t3_ref[...],
                   preferred_element_type=jnp.float32)

    @pl.when(k == 0)
    def _init():
        state_ref[...] = part

    @pl.when(k > 0)
    def _acc():
        state_ref[...] += part


def kernel(adj, features, node_labels, node, attention,
           theta_step_1, theta_step_2, theta_step_3):
    key = jax.random.key(42)
    k1, k2 = jax.random.split(key)
    u1 = jax.random.uniform(k1, ())
    scores = jax.random.uniform(k2, (N,))

    node_arr = jnp.asarray(node, jnp.int32).reshape((1,))

    grid_spec = pltpu.PrefetchScalarGridSpec(
        num_scalar_prefetch=1,
        grid=(NK,),
        in_specs=[
            pl.BlockSpec((8, N), lambda k, n: (n[0] // 8, 0)),
            pl.BlockSpec((1, N), lambda k, n: (0, 0)),
            pl.BlockSpec((1, N), lambda k, n: (0, 0)),
            pl.BlockSpec((1, L), lambda k, n: (0, 0)),
            pl.BlockSpec((1, N), lambda k, n: (0, 0)),
            pl.BlockSpec((1, 1), lambda k, n: (0, 0)),
            pl.BlockSpec((L, STEP_DIM), lambda k, n: (0, 0)),
            pl.BlockSpec((L, STEP_DIM), lambda k, n: (0, 0)),
            pl.BlockSpec((BK, COMB_DIM), lambda k, n: (k, 0)),
        ],
        out_specs=[
            pl.BlockSpec((1, COMB_DIM), lambda k, n: (0, 0)),
            pl.BlockSpec((1, 1), lambda k, n: (0, 0)),
            pl.BlockSpec((1, 1), lambda k, n: (0, 0)),
        ],
        scratch_shapes=[pltpu.VMEM((NK, BK), jnp.float32)],
    )

    state, new_node, att_score = pl.pallas_call(
        _step_kernel,
        grid_spec=grid_spec,
        out_shape=[
            jax.ShapeDtypeStruct((1, COMB_DIM), jnp.float32),
            jax.ShapeDtypeStruct((1, 1), jnp.int32),
            jax.ShapeDtypeStruct((1, 1), jnp.float32),
        ],
    )(node_arr, adj, features.reshape(1, N),
      node_labels.astype(jnp.int32).reshape(1, N),
      attention.reshape(1, L), scores.reshape(1, N), u1.reshape(1, 1),
      theta_step_1, theta_step_2, theta_step_3)

    return (state.reshape(1, 1, COMB_DIM),
            new_node.reshape(()),
            att_score.reshape(()))


# speedup vs baseline: 3.9749x; 1.0224x over previous
import jax
import jax.numpy as jnp
from jax.experimental import pallas as pl
from jax.experimental.pallas import tpu as pltpu

N = 4096
L = 64
STEP_DIM = 2048
COMB_DIM = 1024
BK = 1024
NK = 2 * STEP_DIM // BK
NSTEP = NK // 2
JH = STEP_DIM // BK


def _step_kernel(node_sref, adj_row_ref, feats_ref, labels_ref, att_ref,
                 scores_ref, u_ref, t1_ref, t2_ref, t3a_ref, t3b_ref,
                 state_ref, newnode_ref, attscore_ref, comb_ref):
    k = pl.program_id(0)

    @pl.when(k == 0)
    def _sampling():
        adj_row = adj_row_ref[node_sref[0] % 8, :]
        feats = feats_ref[0, :]
        att = att_ref[0, :]

        nf = jnp.sum(adj_row * feats)
        spread = att * nf
        s = jnp.sum(spread)
        norm = spread / s

        row_ids = jax.lax.broadcasted_iota(jnp.int32, (L, L), 0)
        col_ids = jax.lax.broadcasted_iota(jnp.int32, (L, L), 1)
        tri = (row_ids <= col_ids).astype(jnp.float32)
        cum = jnp.dot(norm.reshape(1, L), tri,
                      preferred_element_type=jnp.float32)[0, :]
        r = cum[L - 1] * (1.0 - u_ref[0, 0])
        label = jnp.sum((cum < r).astype(jnp.int32)).astype(jnp.int32)

        cand = (adj_row > 0.0) & (labels_ref[0, :] == label)
        n_cand = jnp.sum(cand.astype(jnp.int32))
        sc = jnp.where(cand, scores_ref[0, :], -jnp.inf)
        m = jnp.max(sc)
        idx = jax.lax.broadcasted_iota(jnp.int32, (N,), 0)
        first_max = jnp.min(jnp.where(sc == m, idx, N))
        new_node = jnp.where(n_cand > 0, first_max, node_sref[0])
        newnode_ref[...] = new_node.astype(jnp.int32).reshape(1, 1)

        onehot = (jax.lax.broadcasted_iota(jnp.int32, (1, L), 1)
                  == label).astype(jnp.float32)
        attscore_ref[...] = jnp.sum(onehot[0, :] * att).reshape(1, 1)

        att_row = att.reshape(1, L)
        for j in range(JH):
            comb_ref[j:j + 1, :] = jnp.dot(
                att_row, t1_ref[:, j * BK:(j + 1) * BK],
                preferred_element_type=jnp.float32)
        for j in range(JH):
            comb_ref[JH + j:JH + j + 1, :] = jnp.dot(
                onehot, t2_ref[:, j * BK:(j + 1) * BK],
                preferred_element_type=jnp.float32)

    part = jnp.dot(comb_ref[pl.ds(k, 1), :], t3a_ref[...],
                   preferred_element_type=jnp.float32)
    part += jnp.dot(comb_ref[pl.ds(k + NSTEP, 1), :], t3b_ref[...],
                    preferred_element_type=jnp.float32)

    @pl.when(k == 0)
    def _init():
        state_ref[...] = part

    @pl.when(k > 0)
    def _acc():
        state_ref[...] += part


def kernel(adj, features, node_labels, node, attention,
           theta_step_1, theta_step_2, theta_step_3):
    key = jax.random.key(42)
    k1, k2 = jax.random.split(key)
    u1 = jax.random.uniform(k1, ())
    scores = jax.random.uniform(k2, (N,))

    node_arr = jnp.asarray(node, jnp.int32).reshape((1,))

    grid_spec = pltpu.PrefetchScalarGridSpec(
        num_scalar_prefetch=1,
        grid=(NSTEP,),
        in_specs=[
            pl.BlockSpec((8, N), lambda k, n: (n[0] // 8, 0)),
            pl.BlockSpec((1, N), lambda k, n: (0, 0)),
            pl.BlockSpec((1, N), lambda k, n: (0, 0)),
            pl.BlockSpec((1, L), lambda k, n: (0, 0)),
            pl.BlockSpec((1, N), lambda k, n: (0, 0)),
            pl.BlockSpec((1, 1), lambda k, n: (0, 0)),
            pl.BlockSpec((L, STEP_DIM), lambda k, n: (0, 0)),
            pl.BlockSpec((L, STEP_DIM), lambda k, n: (0, 0)),
            pl.BlockSpec((BK, COMB_DIM), lambda k, n: (k, 0)),
            pl.BlockSpec((BK, COMB_DIM), lambda k, n: (k + NSTEP, 0)),
        ],
        out_specs=[
            pl.BlockSpec((1, COMB_DIM), lambda k, n: (0, 0)),
            pl.BlockSpec((1, 1), lambda k, n: (0, 0)),
            pl.BlockSpec((1, 1), lambda k, n: (0, 0)),
        ],
        scratch_shapes=[pltpu.VMEM((NK, BK), jnp.float32)],
    )

    state, new_node, att_score = pl.pallas_call(
        _step_kernel,
        grid_spec=grid_spec,
        out_shape=[
            jax.ShapeDtypeStruct((1, COMB_DIM), jnp.float32),
            jax.ShapeDtypeStruct((1, 1), jnp.int32),
            jax.ShapeDtypeStruct((1, 1), jnp.float32),
        ],
    )(node_arr, adj, features.reshape(1, N),
      node_labels.astype(jnp.int32).reshape(1, N),
      attention.reshape(1, L), scores.reshape(1, N), u1.reshape(1, 1),
      theta_step_1, theta_step_2, theta_step_3, theta_step_3)

    return (state.reshape(1, 1, COMB_DIM),
            new_node.reshape(()),
            att_score.reshape(()))
